# Initial kernel scaffold; baseline (speedup 1.0000x reference)
import jax
import jax.numpy as jnp
from jax.experimental import pallas as pl


def kernel(t, pe):
    raise NotImplementedError("write your pallas kernel here")


# speedup vs baseline: 1.6965x; 1.6965x over previous
import functools

import jax
import jax.numpy as jnp
from jax import lax
from jax.experimental import pallas as pl
from jax.experimental.pallas import tpu as pltpu
from jax.experimental.pallas import tpu_sc as plsc

TIME_STEPS = 1000
EMBED_DIM = 512
BATCH = 16384

NUM_CORES = 2
NUM_SUBCORES = 16
NUM_WORKERS = NUM_CORES * NUM_SUBCORES
ROWS_PER_WORKER = BATCH // NUM_WORKERS
CHUNK = 128
NUM_CHUNKS = ROWS_PER_WORKER // CHUNK


@functools.partial(
    pl.kernel,
    out_type=jax.ShapeDtypeStruct((BATCH, EMBED_DIM), jnp.float32),
    mesh=plsc.VectorSubcoreMesh(core_axis_name="c", subcore_axis_name="s"),
    scratch_types=[
        pltpu.VMEM((NUM_CHUNKS, CHUNK), jnp.int32),
        pltpu.VMEM((CHUNK, EMBED_DIM), jnp.float32),
        pltpu.SemaphoreType.DMA,
    ],
)
def _gather_kernel(t_hbm, pe_hbm, out_hbm, idx_v, rows_v, sem):
    wid = lax.axis_index("s") * NUM_CORES + lax.axis_index("c")
    base = wid * ROWS_PER_WORKER
    pltpu.sync_copy(t_hbm.at[wid], idx_v)
    for j in range(NUM_CHUNKS):
        pltpu.async_copy(pe_hbm.at[idx_v.at[j]], rows_v, sem).wait()
        pltpu.sync_copy(rows_v, out_hbm.at[pl.ds(base + j * CHUNK, CHUNK)])


def kernel(t, pe):
    t32 = t.astype(jnp.int32).reshape(NUM_WORKERS, NUM_CHUNKS, CHUNK)
    out = _gather_kernel(t32, pe)
    return out.reshape(BATCH, EMBED_DIM, 1, 1)


# speedup vs baseline: 1.7131x; 1.0098x over previous
import functools

import jax
import jax.numpy as jnp
from jax import lax
from jax.experimental import pallas as pl
from jax.experimental.pallas import tpu as pltpu
from jax.experimental.pallas import tpu_sc as plsc

TIME_STEPS = 1000
EMBED_DIM = 512
BATCH = 16384

NUM_CORES = 2
NUM_SUBCORES = 16
NUM_WORKERS = NUM_CORES * NUM_SUBCORES
ROWS_PER_WORKER = BATCH // NUM_WORKERS
CHUNK = 64
NUM_CHUNKS = ROWS_PER_WORKER // CHUNK
NBUF = 2


@functools.partial(
    pl.kernel,
    out_type=jax.ShapeDtypeStruct((BATCH, EMBED_DIM), jnp.float32),
    mesh=plsc.VectorSubcoreMesh(core_axis_name="c", subcore_axis_name="s"),
    scratch_types=[
        pltpu.VMEM((NUM_CHUNKS, CHUNK), jnp.int32),
        pltpu.VMEM((NBUF, CHUNK, EMBED_DIM), jnp.float32),
        pltpu.SemaphoreType.DMA((NBUF,)),
        pltpu.SemaphoreType.DMA((NBUF,)),
    ],
)
def _gather_kernel(t_hbm, pe_hbm, out_hbm, idx_v, rows_v, gsem, osem):
    wid = lax.axis_index("s") * NUM_CORES + lax.axis_index("c")
    base = wid * ROWS_PER_WORKER
    pltpu.sync_copy(t_hbm.at[wid], idx_v)

    gathers = [None] * NUM_CHUNKS
    writes = [None] * NUM_CHUNKS
    gathers[0] = pltpu.async_copy(pe_hbm.at[idx_v.at[0]], rows_v.at[0], gsem.at[0])
    for j in range(NUM_CHUNKS):
        b = j % NBUF
        if j + 1 < NUM_CHUNKS:
            nb = (j + 1) % NBUF
            if j - 1 >= 0:
                writes[j - 1].wait()
            gathers[j + 1] = pltpu.async_copy(
                pe_hbm.at[idx_v.at[j + 1]], rows_v.at[nb], gsem.at[nb]
            )
        gathers[j].wait()
        writes[j] = pltpu.async_copy(
            rows_v.at[b], out_hbm.at[pl.ds(base + j * CHUNK, CHUNK)], osem.at[b]
        )
    writes[NUM_CHUNKS - 2].wait()
    writes[NUM_CHUNKS - 1].wait()


def kernel(t, pe):
    t32 = t.astype(jnp.int32).reshape(NUM_WORKERS, NUM_CHUNKS, CHUNK)
    out = _gather_kernel(t32, pe)
    return out.reshape(BATCH, EMBED_DIM, 1, 1)


# speedup vs baseline: 2.6319x; 1.5363x over previous
import functools

import jax
import jax.numpy as jnp
from jax import lax
from jax.experimental import pallas as pl
from jax.experimental.pallas import tpu as pltpu
from jax.experimental.pallas import tpu_sc as plsc

TIME_STEPS = 1000
EMBED_DIM = 512
BATCH = 16384

LANE = 128
SPLIT = EMBED_DIM // LANE
NUM_ROWS = BATCH * SPLIT

NUM_CORES = 2
NUM_SUBCORES = 16
NUM_WORKERS = NUM_CORES * NUM_SUBCORES
ROWS_PER_WORKER = NUM_ROWS // NUM_WORKERS
CHUNK = 128
NUM_CHUNKS = ROWS_PER_WORKER // CHUNK
NBUF = 2


@functools.partial(
    pl.kernel,
    out_type=jax.ShapeDtypeStruct((NUM_ROWS, LANE), jnp.float32),
    mesh=plsc.VectorSubcoreMesh(core_axis_name="c", subcore_axis_name="s"),
    scratch_types=[
        pltpu.VMEM((NUM_CHUNKS, CHUNK), jnp.int32),
        pltpu.VMEM((NBUF, CHUNK, LANE), jnp.float32),
        pltpu.SemaphoreType.DMA((NBUF,)),
        pltpu.SemaphoreType.DMA((NBUF,)),
    ],
)
def _gather_kernel(g_hbm, pe_hbm, out_hbm, idx_v, rows_v, gsem, osem):
    wid = lax.axis_index("s") * NUM_CORES + lax.axis_index("c")
    base = pl.multiple_of(wid * ROWS_PER_WORKER, CHUNK)
    pltpu.sync_copy(g_hbm.at[wid], idx_v)

    gathers = [None] * NUM_CHUNKS
    writes = [None] * NUM_CHUNKS
    gathers[0] = pltpu.async_copy(pe_hbm.at[idx_v.at[0]], rows_v.at[0], gsem.at[0])
    for j in range(NUM_CHUNKS):
        b = j % NBUF
        if j + 1 < NUM_CHUNKS:
            nb = (j + 1) % NBUF
            if j - 1 >= 0:
                writes[j - 1].wait()
            gathers[j + 1] = pltpu.async_copy(
                pe_hbm.at[idx_v.at[j + 1]], rows_v.at[nb], gsem.at[nb]
            )
        gathers[j].wait()
        writes[j] = pltpu.async_copy(
            rows_v.at[b], out_hbm.at[pl.ds(base + j * CHUNK, CHUNK)], osem.at[b]
        )
    writes[NUM_CHUNKS - 2].wait()
    writes[NUM_CHUNKS - 1].wait()


def kernel(t, pe):
    t32 = t.astype(jnp.int32)
    g = (t32[:, None] * SPLIT + jnp.arange(SPLIT, dtype=jnp.int32)[None, :]).reshape(
        NUM_WORKERS, NUM_CHUNKS, CHUNK
    )
    pe_lin = pe.reshape(TIME_STEPS * SPLIT, LANE)
    out = _gather_kernel(g, pe_lin)
    return out.reshape(BATCH, EMBED_DIM, 1, 1)


# speedup vs baseline: 2.6872x; 1.0210x over previous
import functools

import jax
import jax.numpy as jnp
from jax import lax
from jax.experimental import pallas as pl
from jax.experimental.pallas import tpu as pltpu
from jax.experimental.pallas import tpu_sc as plsc

TIME_STEPS = 1000
EMBED_DIM = 512
BATCH = 16384

LANE = 128
SPLIT = EMBED_DIM // LANE
NUM_ROWS = BATCH * SPLIT

NUM_CORES = 2
NUM_SUBCORES = 16
NUM_WORKERS = NUM_CORES * NUM_SUBCORES
T_PER_WORKER = BATCH // NUM_WORKERS
CHUNK = 64
NUM_CHUNKS = T_PER_WORKER // CHUNK
OUT_ROWS_PER_CHUNK = CHUNK * SPLIT
NBUF = 2


@functools.partial(
    pl.kernel,
    out_type=jax.ShapeDtypeStruct((NUM_ROWS, LANE), jnp.float32),
    mesh=plsc.VectorSubcoreMesh(core_axis_name="c", subcore_axis_name="s"),
    scratch_types=[
        pltpu.VMEM((SPLIT, LANE), jnp.int32),
        pltpu.VMEM((NBUF, CHUNK, SPLIT, LANE), jnp.float32),
        pltpu.SemaphoreType.DMA((NBUF,)),
        pltpu.SemaphoreType.DMA((NBUF,)),
    ],
)
def _gather_kernel(t_hbm, pe_hbm, out_hbm, idx_v, rows_v, gsem, osem):
    wid = lax.axis_index("s") * NUM_CORES + lax.axis_index("c")
    base = pl.multiple_of(wid * T_PER_WORKER * SPLIT, OUT_ROWS_PER_CHUNK)
    pltpu.sync_copy(t_hbm.at[pl.ds(SPLIT * wid, SPLIT)], idx_v)

    def idx_slice(j):
        return idx_v.at[j // 2, pl.ds((j % 2) * CHUNK, CHUNK)]

    gathers = [None] * NUM_CHUNKS
    writes = [None] * NUM_CHUNKS
    gathers[0] = pltpu.async_copy(pe_hbm.at[idx_slice(0)], rows_v.at[0], gsem.at[0])
    for j in range(NUM_CHUNKS):
        b = j % NBUF
        if j + 1 < NUM_CHUNKS:
            nb = (j + 1) % NBUF
            if j - 1 >= 0:
                writes[j - 1].wait()
            gathers[j + 1] = pltpu.async_copy(
                pe_hbm.at[idx_slice(j + 1)], rows_v.at[nb], gsem.at[nb]
            )
        gathers[j].wait()
        writes[j] = pltpu.async_copy(
            rows_v.at[b].reshape(OUT_ROWS_PER_CHUNK, LANE),
            out_hbm.at[pl.ds(base + j * OUT_ROWS_PER_CHUNK, OUT_ROWS_PER_CHUNK)],
            osem.at[b],
        )
    writes[NUM_CHUNKS - 2].wait()
    writes[NUM_CHUNKS - 1].wait()


def kernel(t, pe):
    t2 = t.astype(jnp.int32).reshape(BATCH // LANE, LANE)
    pe3 = pe.reshape(TIME_STEPS, SPLIT, LANE)
    out = _gather_kernel(t2, pe3)
    return out.reshape(BATCH, EMBED_DIM, 1, 1)


# speedup vs baseline: 2.6932x; 1.0023x over previous
import functools

import jax
import jax.numpy as jnp
from jax import lax
from jax.experimental import pallas as pl
from jax.experimental.pallas import tpu as pltpu
from jax.experimental.pallas import tpu_sc as plsc

TIME_STEPS = 1000
EMBED_DIM = 512
BATCH = 16384

LANE = 128
SPLIT = EMBED_DIM // LANE
NUM_ROWS = BATCH * SPLIT

NUM_CORES = 2
NUM_SUBCORES = 16
NUM_WORKERS = NUM_CORES * NUM_SUBCORES
T_PER_WORKER = BATCH // NUM_WORKERS
CHUNK = 64
NUM_CHUNKS = T_PER_WORKER // CHUNK
OUT_ROWS_PER_CHUNK = CHUNK * SPLIT
NBUF = 3


@functools.partial(
    pl.kernel,
    out_type=jax.ShapeDtypeStruct((NUM_ROWS, LANE), jnp.float32),
    mesh=plsc.VectorSubcoreMesh(core_axis_name="c", subcore_axis_name="s"),
    scratch_types=[
        pltpu.VMEM((SPLIT, LANE), jnp.int32),
        pltpu.VMEM((NBUF, CHUNK, SPLIT, LANE), jnp.float32),
        pltpu.SemaphoreType.DMA((NBUF,)),
        pltpu.SemaphoreType.DMA((NBUF,)),
    ],
)
def _gather_kernel(t_hbm, pe_hbm, out_hbm, idx_v, rows_v, gsem, osem):
    wid = lax.axis_index("s") * NUM_CORES + lax.axis_index("c")
    base = pl.multiple_of(wid * T_PER_WORKER * SPLIT, OUT_ROWS_PER_CHUNK)
    pltpu.sync_copy(t_hbm.at[pl.ds(SPLIT * wid, SPLIT)], idx_v)

    def idx_slice(j):
        return idx_v.at[j // 2, pl.ds((j % 2) * CHUNK, CHUNK)]

    gathers = [None] * NUM_CHUNKS
    writes = [None] * NUM_CHUNKS
    for j in range(min(NBUF, NUM_CHUNKS)):
        gathers[j] = pltpu.async_copy(
            pe_hbm.at[idx_slice(j)], rows_v.at[j % NBUF], gsem.at[j % NBUF]
        )
    for j in range(NUM_CHUNKS):
        b = j % NBUF
        gathers[j].wait()
        writes[j] = pltpu.async_copy(
            rows_v.at[b].reshape(OUT_ROWS_PER_CHUNK, LANE),
            out_hbm.at[pl.ds(base + j * OUT_ROWS_PER_CHUNK, OUT_ROWS_PER_CHUNK)],
            osem.at[b],
        )
        nxt = j + NBUF
        if nxt < NUM_CHUNKS:
            writes[j].wait()
            gathers[nxt] = pltpu.async_copy(
                pe_hbm.at[idx_slice(nxt)], rows_v.at[b], gsem.at[b]
            )
    for j in range(max(0, NUM_CHUNKS - NBUF), NUM_CHUNKS):
        writes[j].wait()


def kernel(t, pe):
    t2 = t.astype(jnp.int32).reshape(BATCH // LANE, LANE)
    pe3 = pe.reshape(TIME_STEPS, SPLIT, LANE)
    out = _gather_kernel(t2, pe3)
    return out.reshape(BATCH, EMBED_DIM, 1, 1)


# speedup vs baseline: 3.3876x; 1.2578x over previous
import functools

import jax
import jax.numpy as jnp
from jax import lax
from jax.experimental import pallas as pl
from jax.experimental.pallas import tpu as pltpu
from jax.experimental.pallas import tpu_sc as plsc

TIME_STEPS = 1000
EMBED_DIM = 512
BATCH = 16384

LANE = 128
SPLIT = EMBED_DIM // LANE
NUM_ROWS = BATCH * SPLIT
TBL_ROWS = TIME_STEPS * SPLIT

NUM_CORES = 2
NUM_SUBCORES = 16
NUM_WORKERS = NUM_CORES * NUM_SUBCORES
T_PER_WORKER = BATCH // NUM_WORKERS
CHUNK = 64
NUM_CHUNKS = T_PER_WORKER // CHUNK
OUT_ROWS_PER_CHUNK = CHUNK * SPLIT
NBUF = 2


@functools.partial(
    pl.kernel,
    out_type=jax.ShapeDtypeStruct((NUM_ROWS, LANE), jnp.float32),
    mesh=plsc.VectorSubcoreMesh(core_axis_name="c", subcore_axis_name="s"),
    scratch_types=[
        pltpu.VMEM((SPLIT, LANE), jnp.int32),
        pltpu.VMEM((NBUF, CHUNK, SPLIT, LANE), jnp.float32),
        pltpu.VMEM_SHARED((TIME_STEPS, SPLIT, LANE), jnp.float32),
        pltpu.SemaphoreType.DMA((NBUF,)),
        pltpu.SemaphoreType.DMA((NBUF,)),
    ],
)
def _gather_kernel(t_hbm, pe_hbm, out_hbm, idx_v, rows_v, pe_sh, gsem, osem):
    wid = lax.axis_index("s") * NUM_CORES + lax.axis_index("c")
    sid = lax.axis_index("s")
    base = pl.multiple_of(wid * T_PER_WORKER * SPLIT, OUT_ROWS_PER_CHUNK)

    srow = jnp.minimum(sid * 63, TIME_STEPS - 63)
    pltpu.sync_copy(
        pe_hbm.at[pl.ds(srow, 63)],
        pe_sh.at[pl.ds(srow, 63)],
    )
    pltpu.sync_copy(t_hbm.at[pl.ds(SPLIT * wid, SPLIT)], idx_v)
    plsc.subcore_barrier()

    def idx_slice(j):
        return idx_v.at[j // 2, pl.ds((j % 2) * CHUNK, CHUNK)]

    gathers = [None] * NUM_CHUNKS
    writes = [None] * NUM_CHUNKS
    for j in range(min(NBUF, NUM_CHUNKS)):
        gathers[j] = pltpu.async_copy(
            pe_sh.at[idx_slice(j)], rows_v.at[j % NBUF], gsem.at[j % NBUF]
        )
    for j in range(NUM_CHUNKS):
        b = j % NBUF
        gathers[j].wait()
        writes[j] = pltpu.async_copy(
            rows_v.at[b].reshape(OUT_ROWS_PER_CHUNK, LANE),
            out_hbm.at[pl.ds(base + j * OUT_ROWS_PER_CHUNK, OUT_ROWS_PER_CHUNK)],
            osem.at[b],
        )
        nxt = j + NBUF
        if nxt < NUM_CHUNKS:
            writes[j].wait()
            gathers[nxt] = pltpu.async_copy(
                pe_sh.at[idx_slice(nxt)], rows_v.at[b], gsem.at[b]
            )
    for j in range(max(0, NUM_CHUNKS - NBUF), NUM_CHUNKS):
        writes[j].wait()


def kernel(t, pe):
    t2 = t.astype(jnp.int32).reshape(BATCH // LANE, LANE)
    pe3 = pe.reshape(TIME_STEPS, SPLIT, LANE)
    out = _gather_kernel(t2, pe3)
    return out.reshape(BATCH, EMBED_DIM, 1, 1)


# speedup vs baseline: 3.4567x; 1.0204x over previous
import functools

import jax
import jax.numpy as jnp
from jax import lax
from jax.experimental import pallas as pl
from jax.experimental.pallas import tpu as pltpu
from jax.experimental.pallas import tpu_sc as plsc

TIME_STEPS = 1000
EMBED_DIM = 512
BATCH = 16384

LANE = 128
SPLIT = EMBED_DIM // LANE
NUM_ROWS = BATCH * SPLIT
TBL_ROWS = TIME_STEPS * SPLIT

NUM_CORES = 2
NUM_SUBCORES = 16
NUM_WORKERS = NUM_CORES * NUM_SUBCORES
T_PER_WORKER = BATCH // NUM_WORKERS
CHUNK = 64
NUM_CHUNKS = T_PER_WORKER // CHUNK
OUT_ROWS_PER_CHUNK = CHUNK * SPLIT
NBUF = 2


@functools.partial(
    pl.kernel,
    out_type=jax.ShapeDtypeStruct((NUM_ROWS, LANE), jnp.float32),
    mesh=plsc.VectorSubcoreMesh(core_axis_name="c", subcore_axis_name="s"),
    scratch_types=[
        pltpu.VMEM((SPLIT, LANE), jnp.int32),
        pltpu.VMEM((NBUF, CHUNK, SPLIT, LANE), jnp.float32),
        pltpu.VMEM_SHARED((TIME_STEPS, SPLIT, LANE), jnp.float32),
        pltpu.SemaphoreType.DMA((NBUF,)),
        pltpu.SemaphoreType.DMA((NBUF,)),
    ],
)
def _gather_kernel(t_hbm, pe_hbm, out_hbm, idx_v, rows_v, pe_sh, gsem, osem):
    wid = lax.axis_index("s") * NUM_CORES + lax.axis_index("c")
    sid = lax.axis_index("s")
    base = pl.multiple_of(wid * T_PER_WORKER * SPLIT, OUT_ROWS_PER_CHUNK)

    pltpu.sync_copy(t_hbm.at[pl.ds(SPLIT * wid, SPLIT)], idx_v)

    def idx_slice(j):
        return idx_v.at[j // 2, pl.ds((j % 2) * CHUNK, CHUNK)]

    gathers = [None] * NUM_CHUNKS
    writes = [None] * NUM_CHUNKS
    for j in range(min(NBUF, NUM_CHUNKS)):
        gathers[j] = pltpu.async_copy(
            pe_hbm.at[idx_slice(j)], rows_v.at[j % NBUF], gsem.at[j % NBUF]
        )

    srow = jnp.minimum(sid * 63, TIME_STEPS - 63)
    pltpu.sync_copy(
        pe_hbm.at[pl.ds(srow, 63)],
        pe_sh.at[pl.ds(srow, 63)],
    )
    plsc.subcore_barrier()
    for j in range(NUM_CHUNKS):
        b = j % NBUF
        gathers[j].wait()
        writes[j] = pltpu.async_copy(
            rows_v.at[b].reshape(OUT_ROWS_PER_CHUNK, LANE),
            out_hbm.at[pl.ds(base + j * OUT_ROWS_PER_CHUNK, OUT_ROWS_PER_CHUNK)],
            osem.at[b],
        )
        nxt = j + NBUF
        if nxt < NUM_CHUNKS:
            writes[j].wait()
            gathers[nxt] = pltpu.async_copy(
                pe_sh.at[idx_slice(nxt)], rows_v.at[b], gsem.at[b]
            )
    for j in range(max(0, NUM_CHUNKS - NBUF), NUM_CHUNKS):
        writes[j].wait()


def kernel(t, pe):
    t2 = t.astype(jnp.int32).reshape(BATCH // LANE, LANE)
    pe3 = pe.reshape(TIME_STEPS, SPLIT, LANE)
    out = _gather_kernel(t2, pe3)
    return out.reshape(BATCH, EMBED_DIM, 1, 1)
